# Initial kernel scaffold; baseline (speedup 1.0000x reference)
import jax
import jax.numpy as jnp
from jax.experimental import pallas as pl


def kernel(hidden_states, W):
    raise NotImplementedError("write your pallas kernel here")


# speedup vs baseline: 2.9549x; 2.9549x over previous
import functools

import jax
import jax.numpy as jnp
from jax.experimental import pallas as pl
from jax.experimental.pallas import tpu as pltpu

HIDDEN = 7168
NUM_EXPERTS = 256
TOP_K = 8
N_GROUP = 8
TOPK_GROUP = 4
EPG = NUM_EXPERTS // N_GROUP

TOKEN_BLOCK = 256


def _router_kernel(x_ref, w_ref, wts_ref, idx_ref, logits_ref):
    x = x_ref[...].astype(jnp.bfloat16)
    w = w_ref[...].astype(jnp.bfloat16)
    logits = jax.lax.dot_general(
        x, w, (((1,), (1,)), ((), ())), preferred_element_type=jnp.float32,
    )
    logits_ref[...] = logits

    scores = jax.nn.sigmoid(logits)

    row_g = jax.lax.broadcasted_iota(jnp.int32, (NUM_EXPERTS, NUM_EXPERTS), 0) // EPG
    col_g = jax.lax.broadcasted_iota(jnp.int32, (NUM_EXPERTS, NUM_EXPERTS), 1) // EPG
    gmat = (row_g == col_g).astype(jnp.float32)
    gsum = jax.lax.dot_general(
        scores, gmat, (((1,), (0,)), ((), ())), preferred_element_type=jnp.float32,
        precision=jax.lax.Precision.HIGHEST,
    )

    g = gsum
    chosen = jnp.zeros(g.shape, dtype=jnp.bool_)
    for _ in range(TOPK_GROUP):
        m = jnp.max(g, axis=-1, keepdims=True)
        sel = g == m
        chosen = jnp.logical_or(chosen, sel)
        g = jnp.where(sel, -jnp.inf, g)

    masked = jnp.where(chosen, scores, 0.0)

    lane = jax.lax.broadcasted_iota(jnp.int32, masked.shape, 1)
    wcols = []
    icols = []
    for _ in range(TOP_K):
        m = jnp.max(masked, axis=-1, keepdims=True)
        is_m = masked == m
        idx = jnp.min(jnp.where(is_m, lane, NUM_EXPERTS), axis=-1, keepdims=True)
        wcols.append(m)
        icols.append(idx)
        masked = jnp.where(lane == idx, -1.0, masked)

    wts = jnp.concatenate(wcols, axis=-1)
    idx = jnp.concatenate(icols, axis=-1)
    denom = jnp.clip(jnp.sum(wts, axis=-1, keepdims=True), 1e-12, None)
    wts_ref[...] = wts / denom
    idx_ref[...] = idx


@jax.jit
def kernel(hidden_states, W):
    T = hidden_states.shape[0]
    grid = (T // TOKEN_BLOCK,)
    wts, idx, logits = pl.pallas_call(
        _router_kernel,
        grid=grid,
        in_specs=[
            pl.BlockSpec((TOKEN_BLOCK, HIDDEN), lambda i: (i, 0)),
            pl.BlockSpec((NUM_EXPERTS, HIDDEN), lambda i: (0, 0)),
        ],
        out_specs=[
            pl.BlockSpec((TOKEN_BLOCK, TOP_K), lambda i: (i, 0)),
            pl.BlockSpec((TOKEN_BLOCK, TOP_K), lambda i: (i, 0)),
            pl.BlockSpec((TOKEN_BLOCK, NUM_EXPERTS), lambda i: (i, 0)),
        ],
        out_shape=[
            jax.ShapeDtypeStruct((T, TOP_K), jnp.float32),
            jax.ShapeDtypeStruct((T, TOP_K), jnp.int32),
            jax.ShapeDtypeStruct((T, NUM_EXPERTS), jnp.float32),
        ],
        compiler_params=pltpu.CompilerParams(
            dimension_semantics=("arbitrary",),
        ),
    )(hidden_states, W)
    return wts, idx.astype(jnp.int64), logits


# speedup vs baseline: 3.2135x; 1.0875x over previous
import functools

import jax
import jax.numpy as jnp
from jax.experimental import pallas as pl
from jax.experimental.pallas import tpu as pltpu

HIDDEN = 7168
NUM_EXPERTS = 256
TOP_K = 8
N_GROUP = 8
TOPK_GROUP = 4
EPG = NUM_EXPERTS // N_GROUP

TOKEN_BLOCK = 256


def _router_kernel(x_ref, w_ref, wts_ref, idx_ref, logits_ref):
    x = x_ref[...].astype(jnp.bfloat16)
    w = w_ref[...]
    logits = jax.lax.dot_general(
        x, w, (((1,), (1,)), ((), ())), preferred_element_type=jnp.float32,
    )
    logits_ref[...] = logits

    scores = jax.nn.sigmoid(logits)

    row_g = jax.lax.broadcasted_iota(jnp.int32, (NUM_EXPERTS, NUM_EXPERTS), 0) // EPG
    col_g = jax.lax.broadcasted_iota(jnp.int32, (NUM_EXPERTS, NUM_EXPERTS), 1) // EPG
    gmat = (row_g == col_g).astype(jnp.float32)
    gsum = jax.lax.dot_general(
        scores, gmat, (((1,), (0,)), ((), ())), preferred_element_type=jnp.float32,
        precision=jax.lax.Precision.HIGHEST,
    )

    g = gsum
    chosen = jnp.zeros(g.shape, dtype=jnp.bool_)
    for _ in range(TOPK_GROUP):
        m = jnp.max(g, axis=-1, keepdims=True)
        sel = g == m
        chosen = jnp.logical_or(chosen, sel)
        g = jnp.where(sel, -jnp.inf, g)

    masked = jnp.where(chosen, scores, 0.0)

    lane = jax.lax.broadcasted_iota(jnp.int32, masked.shape, 1).astype(jnp.float32)
    wcols = []
    icols = []
    for _ in range(TOP_K):
        m = jnp.max(masked, axis=-1, keepdims=True)
        is_m = masked == m
        idx = jnp.min(jnp.where(is_m, lane, float(NUM_EXPERTS)), axis=-1,
                      keepdims=True)
        wcols.append(m)
        icols.append(idx)
        masked = jnp.where(lane == idx, -1.0, masked)

    wts = jnp.concatenate(wcols, axis=-1)
    idx = jnp.concatenate(icols, axis=-1)
    denom = jnp.clip(jnp.sum(wts, axis=-1, keepdims=True), 1e-12, None)
    wts_ref[...] = wts / denom
    idx_ref[...] = idx.astype(jnp.int32)


@jax.jit
def kernel(hidden_states, W):
    T = hidden_states.shape[0]
    grid = (T // TOKEN_BLOCK,)
    w_bf16 = W.astype(jnp.bfloat16)
    wts, idx, logits = pl.pallas_call(
        _router_kernel,
        grid=grid,
        in_specs=[
            pl.BlockSpec((TOKEN_BLOCK, HIDDEN), lambda i: (i, 0)),
            pl.BlockSpec((NUM_EXPERTS, HIDDEN), lambda i: (0, 0)),
        ],
        out_specs=[
            pl.BlockSpec((TOKEN_BLOCK, TOP_K), lambda i: (i, 0)),
            pl.BlockSpec((TOKEN_BLOCK, TOP_K), lambda i: (i, 0)),
            pl.BlockSpec((TOKEN_BLOCK, NUM_EXPERTS), lambda i: (i, 0)),
        ],
        out_shape=[
            jax.ShapeDtypeStruct((T, TOP_K), jnp.float32),
            jax.ShapeDtypeStruct((T, TOP_K), jnp.int32),
            jax.ShapeDtypeStruct((T, NUM_EXPERTS), jnp.float32),
        ],
        compiler_params=pltpu.CompilerParams(
            dimension_semantics=("arbitrary",),
        ),
    )(hidden_states, w_bf16)
    return wts, idx.astype(jnp.int64), logits


# speedup vs baseline: 3.7977x; 1.1818x over previous
import functools

import jax
import jax.numpy as jnp
from jax.experimental import pallas as pl
from jax.experimental.pallas import tpu as pltpu

HIDDEN = 7168
NUM_EXPERTS = 256
TOP_K = 8
N_GROUP = 8
TOPK_GROUP = 4
EPG = NUM_EXPERTS // N_GROUP

TOKEN_BLOCK = 512


def _router_kernel(x_ref, w_ref, wts_ref, idx_ref, logits_ref):
    x = x_ref[...].astype(jnp.bfloat16)
    w = w_ref[...]
    logits = jax.lax.dot_general(
        x, w, (((1,), (1,)), ((), ())), preferred_element_type=jnp.float32,
    )
    logits_ref[...] = logits

    scores = jax.nn.sigmoid(logits)

    row_g = jax.lax.broadcasted_iota(jnp.int32, (NUM_EXPERTS, NUM_EXPERTS), 0) // EPG
    col_g = jax.lax.broadcasted_iota(jnp.int32, (NUM_EXPERTS, NUM_EXPERTS), 1) // EPG
    gmat = (row_g == col_g).astype(jnp.float32)
    gsum = jax.lax.dot_general(
        scores, gmat, (((1,), (0,)), ((), ())), preferred_element_type=jnp.float32,
        precision=jax.lax.Precision.HIGHEST,
    )

    g = gsum
    chosen = jnp.zeros(g.shape, dtype=jnp.bool_)
    for _ in range(TOPK_GROUP):
        m = jnp.max(g, axis=-1, keepdims=True)
        sel = g == m
        chosen = jnp.logical_or(chosen, sel)
        g = jnp.where(sel, -jnp.inf, g)

    masked = jnp.where(chosen, scores, 0.0)

    lane = jax.lax.broadcasted_iota(jnp.int32, masked.shape, 1).astype(jnp.float32)
    wcols = []
    icols = []
    for _ in range(TOP_K):
        m = jnp.max(masked, axis=-1, keepdims=True)
        is_m = masked == m
        idx = jnp.min(jnp.where(is_m, lane, float(NUM_EXPERTS)), axis=-1,
                      keepdims=True)
        wcols.append(m)
        icols.append(idx)
        masked = jnp.where(lane == idx, -1.0, masked)

    wts = jnp.concatenate(wcols, axis=-1)
    idx = jnp.concatenate(icols, axis=-1)
    denom = jnp.clip(jnp.sum(wts, axis=-1, keepdims=True), 1e-12, None)
    wts_ref[...] = wts / denom
    idx_ref[...] = idx.astype(jnp.int32)


@jax.jit
def kernel(hidden_states, W):
    T = hidden_states.shape[0]
    grid = (T // TOKEN_BLOCK,)
    w_bf16 = W.astype(jnp.bfloat16)
    wts, idx, logits = pl.pallas_call(
        _router_kernel,
        grid=grid,
        in_specs=[
            pl.BlockSpec((TOKEN_BLOCK, HIDDEN), lambda i: (i, 0)),
            pl.BlockSpec((NUM_EXPERTS, HIDDEN), lambda i: (0, 0)),
        ],
        out_specs=[
            pl.BlockSpec((TOKEN_BLOCK, TOP_K), lambda i: (i, 0)),
            pl.BlockSpec((TOKEN_BLOCK, TOP_K), lambda i: (i, 0)),
            pl.BlockSpec((TOKEN_BLOCK, NUM_EXPERTS), lambda i: (i, 0)),
        ],
        out_shape=[
            jax.ShapeDtypeStruct((T, TOP_K), jnp.float32),
            jax.ShapeDtypeStruct((T, TOP_K), jnp.int32),
            jax.ShapeDtypeStruct((T, NUM_EXPERTS), jnp.float32),
        ],
        compiler_params=pltpu.CompilerParams(
            dimension_semantics=("arbitrary",),
        ),
    )(hidden_states, w_bf16)
    return wts, idx.astype(jnp.int64), logits


# speedup vs baseline: 4.0757x; 1.0732x over previous
import functools

import jax
import jax.numpy as jnp
from jax.experimental import pallas as pl
from jax.experimental.pallas import tpu as pltpu

HIDDEN = 7168
NUM_EXPERTS = 256
TOP_K = 8
N_GROUP = 8
TOPK_GROUP = 4
EPG = NUM_EXPERTS // N_GROUP

TOKEN_BLOCK = 512


def _router_kernel(x_ref, w_ref, wts_ref, idx_ref, logits_ref):
    x = x_ref[...].astype(jnp.bfloat16)
    w = w_ref[...].astype(jnp.bfloat16)
    logits = jax.lax.dot_general(
        x, w, (((1,), (1,)), ((), ())), preferred_element_type=jnp.float32,
    )
    logits_ref[...] = logits

    scores = jax.nn.sigmoid(logits)

    row_g = jax.lax.broadcasted_iota(jnp.int32, (NUM_EXPERTS, NUM_EXPERTS), 0) // EPG
    col_g = jax.lax.broadcasted_iota(jnp.int32, (NUM_EXPERTS, NUM_EXPERTS), 1) // EPG
    gmat = (row_g == col_g).astype(jnp.float32)
    gsum = jax.lax.dot_general(
        scores, gmat, (((1,), (0,)), ((), ())), preferred_element_type=jnp.float32,
        precision=jax.lax.Precision.HIGHEST,
    )

    g = gsum
    chosen = jnp.zeros(g.shape, dtype=jnp.bool_)
    for _ in range(TOPK_GROUP):
        m = jnp.max(g, axis=-1, keepdims=True)
        sel = g == m
        chosen = jnp.logical_or(chosen, sel)
        g = jnp.where(sel, -jnp.inf, g)

    masked = jnp.where(chosen, scores, 0.0)

    lane = jax.lax.broadcasted_iota(jnp.int32, masked.shape, 1).astype(jnp.float32)
    wcols = []
    icols = []
    for _ in range(TOP_K):
        m = jnp.max(masked, axis=-1, keepdims=True)
        is_m = masked == m
        idx = jnp.min(jnp.where(is_m, lane, float(NUM_EXPERTS)), axis=-1,
                      keepdims=True)
        wcols.append(m)
        icols.append(idx)
        masked = jnp.where(lane == idx, -1.0, masked)

    wts = jnp.concatenate(wcols, axis=-1)
    idx = jnp.concatenate(icols, axis=-1)
    denom = jnp.clip(jnp.sum(wts, axis=-1, keepdims=True), 1e-12, None)
    wts_ref[...] = wts / denom
    idx_ref[...] = idx.astype(jnp.int32)


@jax.jit
def kernel(hidden_states, W):
    T = hidden_states.shape[0]
    grid = (T // TOKEN_BLOCK,)
    wts, idx, logits = pl.pallas_call(
        _router_kernel,
        grid=grid,
        in_specs=[
            pl.BlockSpec((TOKEN_BLOCK, HIDDEN), lambda i: (i, 0)),
            pl.BlockSpec((NUM_EXPERTS, HIDDEN), lambda i: (0, 0)),
        ],
        out_specs=[
            pl.BlockSpec((TOKEN_BLOCK, TOP_K), lambda i: (i, 0)),
            pl.BlockSpec((TOKEN_BLOCK, TOP_K), lambda i: (i, 0)),
            pl.BlockSpec((TOKEN_BLOCK, NUM_EXPERTS), lambda i: (i, 0)),
        ],
        out_shape=[
            jax.ShapeDtypeStruct((T, TOP_K), jnp.float32),
            jax.ShapeDtypeStruct((T, TOP_K), jnp.int32),
            jax.ShapeDtypeStruct((T, NUM_EXPERTS), jnp.float32),
        ],
        compiler_params=pltpu.CompilerParams(
            dimension_semantics=("arbitrary",),
        ),
    )(hidden_states, W)
    return wts, idx.astype(jnp.int64), logits


# speedup vs baseline: 4.1325x; 1.0139x over previous
import functools

import jax
import jax.numpy as jnp
from jax.experimental import pallas as pl
from jax.experimental.pallas import tpu as pltpu

HIDDEN = 7168
NUM_EXPERTS = 256
TOP_K = 8
N_GROUP = 8
TOPK_GROUP = 4
EPG = NUM_EXPERTS // N_GROUP

TOKEN_BLOCK = 512


def _router_kernel(x_ref, w_ref, wts_ref, idx_ref, logits_ref):
    x = x_ref[...].astype(jnp.bfloat16)
    w = w_ref[...].astype(jnp.bfloat16)
    logits = jax.lax.dot_general(
        x, w, (((1,), (1,)), ((), ())), preferred_element_type=jnp.float32,
    )
    logits_ref[...] = logits

    scores = jax.nn.sigmoid(logits)

    row_g = jax.lax.broadcasted_iota(jnp.int32, (NUM_EXPERTS, NUM_EXPERTS), 0) // EPG
    col_g = jax.lax.broadcasted_iota(jnp.int32, (NUM_EXPERTS, NUM_EXPERTS), 1) // EPG
    gmat = (row_g == col_g).astype(jnp.float32)
    gsum = jax.lax.dot_general(
        scores, gmat, (((1,), (0,)), ((), ())), preferred_element_type=jnp.float32,
        precision=jax.lax.Precision.HIGHEST,
    )

    g = gsum
    chosen = jnp.zeros(g.shape, dtype=jnp.bool_)
    for _ in range(TOPK_GROUP):
        m = jnp.max(g, axis=-1, keepdims=True)
        sel = g == m
        chosen = jnp.logical_or(chosen, sel)
        g = jnp.where(sel, -jnp.inf, g)

    masked = jnp.where(chosen, scores, 0.0)

    lane = jax.lax.broadcasted_iota(jnp.int32, masked.shape, 1).astype(jnp.float32)
    wcols = []
    icols = []
    for _ in range(TOP_K):
        m = jnp.max(masked, axis=-1, keepdims=True)
        is_m = masked == m
        idx = jnp.min(jnp.where(is_m, lane, float(NUM_EXPERTS)), axis=-1,
                      keepdims=True)
        wcols.append(m)
        icols.append(idx)
        masked = jnp.where(is_m, -1.0, masked)

    wts = jnp.concatenate(wcols, axis=-1)
    idx = jnp.concatenate(icols, axis=-1)
    denom = jnp.clip(jnp.sum(wts, axis=-1, keepdims=True), 1e-12, None)
    wts_ref[...] = wts / denom
    idx_ref[...] = idx.astype(jnp.int32)


@jax.jit
def kernel(hidden_states, W):
    T = hidden_states.shape[0]
    grid = (T // TOKEN_BLOCK,)
    wts, idx, logits = pl.pallas_call(
        _router_kernel,
        grid=grid,
        in_specs=[
            pl.BlockSpec((TOKEN_BLOCK, HIDDEN), lambda i: (i, 0)),
            pl.BlockSpec((NUM_EXPERTS, HIDDEN), lambda i: (0, 0)),
        ],
        out_specs=[
            pl.BlockSpec((TOKEN_BLOCK, TOP_K), lambda i: (i, 0)),
            pl.BlockSpec((TOKEN_BLOCK, TOP_K), lambda i: (i, 0)),
            pl.BlockSpec((TOKEN_BLOCK, NUM_EXPERTS), lambda i: (i, 0)),
        ],
        out_shape=[
            jax.ShapeDtypeStruct((T, TOP_K), jnp.float32),
            jax.ShapeDtypeStruct((T, TOP_K), jnp.int32),
            jax.ShapeDtypeStruct((T, NUM_EXPERTS), jnp.float32),
        ],
        compiler_params=pltpu.CompilerParams(
            dimension_semantics=("arbitrary",),
        ),
    )(hidden_states, W)
    return wts, idx.astype(jnp.int64), logits
